# Initial kernel scaffold; baseline (speedup 1.0000x reference)
import jax
import jax.numpy as jnp
from jax.experimental import pallas as pl


def kernel(X, keys):
    raise NotImplementedError("write your pallas kernel here")


# speedup vs baseline: 3.6871x; 3.6871x over previous
import functools

import jax
import jax.numpy as jnp
from jax import lax
from jax.experimental import pallas as pl
from jax.experimental.pallas import tpu as pltpu
from jax.experimental.pallas import tpu_sc as plsc

N_ROWS = 320000
D_FEAT = 128
NUM_SEGMENTS = 10000

NC = 2
NS = 16
NW = NC * NS
ROWS_PER_W = N_ROWS // NW
BLK = 80
NBLK = ROWS_PER_W // BLK
OCHUNK = 80
NOCHUNK = NUM_SEGMENTS // OCHUNK
OITER = -(-NOCHUNK // NS)


def _sc_partial_sums(X, keys):
    mesh = plsc.VectorSubcoreMesh(core_axis_name="c", subcore_axis_name="s")

    @functools.partial(
        pl.kernel,
        out_type=jax.ShapeDtypeStruct((NC, NUM_SEGMENTS, D_FEAT), jnp.float32),
        mesh=mesh,
        scratch_types=[
            pltpu.VMEM((BLK, D_FEAT), jnp.float32),
            pltpu.VMEM((BLK,), jnp.int32),
            pltpu.VMEM((OCHUNK, D_FEAT), jnp.float32),
            pltpu.VMEM_SHARED((NUM_SEGMENTS, D_FEAT), jnp.float32),
        ],
    )
    def k(x_hbm, keys_hbm, out_hbm, xbuf, kbuf, zbuf, acc):
        c = lax.axis_index("c")
        s = lax.axis_index("s")
        wid = c * NS + s

        @pl.loop(0, OCHUNK)
        def _(r):
            @pl.loop(0, D_FEAT, step=16)
            def _(col):
                zbuf[r, pl.ds(col, 16)] = jnp.zeros((16,), jnp.float32)

        @pl.loop(0, OITER)
        def _(j):
            chunk = s + NS * j

            @pl.when(chunk < NOCHUNK)
            def _():
                pltpu.sync_copy(zbuf, acc.at[pl.ds(chunk * OCHUNK, OCHUNK)])

        plsc.subcore_barrier()

        base = wid * ROWS_PER_W

        @pl.loop(0, NBLK)
        def _(i):
            off = base + i * BLK
            pltpu.sync_copy(x_hbm.at[pl.ds(off, BLK)], xbuf)
            pltpu.sync_copy(keys_hbm.at[pl.ds(off, BLK)], kbuf)
            pltpu.sync_copy(xbuf, acc.at[kbuf], add=True)

        plsc.subcore_barrier()

        @pl.loop(0, OITER)
        def _(j):
            chunk = s + NS * j

            @pl.when(chunk < NOCHUNK)
            def _():
                pltpu.sync_copy(
                    acc.at[pl.ds(chunk * OCHUNK, OCHUNK)],
                    out_hbm.at[c, pl.ds(chunk * OCHUNK, OCHUNK)],
                )

    return k(X, keys)


def _tc_combine(a, b):
    def body(a_ref, b_ref, o_ref):
        o_ref[...] = a_ref[...] + b_ref[...]

    return pl.pallas_call(
        body,
        grid=(10,),
        in_specs=[
            pl.BlockSpec((1000, D_FEAT), lambda i: (i, 0)),
            pl.BlockSpec((1000, D_FEAT), lambda i: (i, 0)),
        ],
        out_specs=pl.BlockSpec((1000, D_FEAT), lambda i: (i, 0)),
        out_shape=jax.ShapeDtypeStruct((NUM_SEGMENTS, D_FEAT), jnp.float32),
    )(a, b)


@jax.jit
def kernel(X, keys):
    keys = keys.astype(jnp.int32)
    acc = _sc_partial_sums(X, keys)
    return _tc_combine(acc[0], acc[1])


# speedup vs baseline: 5.7921x; 1.5709x over previous
import functools

import jax
import jax.numpy as jnp
from jax import lax
from jax.experimental import pallas as pl
from jax.experimental.pallas import tpu as pltpu
from jax.experimental.pallas import tpu_sc as plsc

N_ROWS = 320000
D_FEAT = 128
NUM_SEGMENTS = 10000

NC = 2
NS = 16
NW = NC * NS
ROWS_PER_W = N_ROWS // NW
BLK = 80
NBLK = ROWS_PER_W // BLK
OCHUNK = 80
NOCHUNK = NUM_SEGMENTS // OCHUNK
OITER = -(-NOCHUNK // NS)


def _sc_partial_sums(X, keys):
    mesh = plsc.VectorSubcoreMesh(core_axis_name="c", subcore_axis_name="s")

    @functools.partial(
        pl.kernel,
        out_type=jax.ShapeDtypeStruct((NC, NUM_SEGMENTS, D_FEAT), jnp.float32),
        mesh=mesh,
        scratch_types=[
            pltpu.VMEM((BLK, D_FEAT), jnp.float32),
            pltpu.VMEM((BLK, D_FEAT), jnp.float32),
            pltpu.VMEM((BLK,), jnp.int32),
            pltpu.VMEM((BLK,), jnp.int32),
            pltpu.VMEM((OCHUNK, D_FEAT), jnp.float32),
            pltpu.VMEM_SHARED((NUM_SEGMENTS, D_FEAT), jnp.float32),
            pltpu.SemaphoreType.DMA,
            pltpu.SemaphoreType.DMA,
        ],
    )
    def k(x_hbm, keys_hbm, out_hbm, xbuf_a, xbuf_b, kbuf_a, kbuf_b,
          zbuf, acc, sem_a, sem_b):
        c = lax.axis_index("c")
        s = lax.axis_index("s")
        wid = c * NS + s

        @pl.loop(0, OCHUNK)
        def _(r):
            @pl.loop(0, D_FEAT, step=16)
            def _(col):
                zbuf[r, pl.ds(col, 16)] = jnp.zeros((16,), jnp.float32)

        @pl.loop(0, OITER)
        def _(j):
            chunk = s + NS * j

            @pl.when(chunk < NOCHUNK)
            def _():
                pltpu.sync_copy(zbuf, acc.at[pl.ds(chunk * OCHUNK, OCHUNK)])

        plsc.subcore_barrier()

        base = wid * ROWS_PER_W

        def kslc(i):
            return keys_hbm.at[pl.ds(base + i * BLK, BLK)]

        def xslc(i):
            return x_hbm.at[pl.ds(base + i * BLK, BLK)]

        def start_load(i, xbuf, kbuf, sem):
            pltpu.async_copy(xslc(i), xbuf, sem)
            pltpu.async_copy(kslc(i), kbuf, sem)

        def wait_load(i, xbuf, kbuf, sem):
            pltpu.make_async_copy(xslc(i), xbuf, sem).wait()
            pltpu.make_async_copy(kslc(i), kbuf, sem).wait()

        start_load(0, xbuf_a, kbuf_a, sem_a)

        @pl.loop(0, NBLK // 2)
        def _(j):
            i0 = 2 * j
            wait_load(i0, xbuf_a, kbuf_a, sem_a)
            start_load(i0 + 1, xbuf_b, kbuf_b, sem_b)
            pltpu.sync_copy(xbuf_a, acc.at[kbuf_a], add=True)
            wait_load(i0 + 1, xbuf_b, kbuf_b, sem_b)
            start_load(i0 + 2, xbuf_a, kbuf_a, sem_a)
            pltpu.sync_copy(xbuf_b, acc.at[kbuf_b], add=True)

        wait_load(NBLK - 1, xbuf_a, kbuf_a, sem_a)
        pltpu.sync_copy(xbuf_a, acc.at[kbuf_a], add=True)

        plsc.subcore_barrier()

        @pl.loop(0, OITER)
        def _(j):
            chunk = s + NS * j

            @pl.when(chunk < NOCHUNK)
            def _():
                pltpu.sync_copy(
                    acc.at[pl.ds(chunk * OCHUNK, OCHUNK)],
                    out_hbm.at[c, pl.ds(chunk * OCHUNK, OCHUNK)],
                )

    return k(X, keys)


def _tc_combine(a, b):
    def body(a_ref, b_ref, o_ref):
        o_ref[...] = a_ref[...] + b_ref[...]

    return pl.pallas_call(
        body,
        grid=(10,),
        in_specs=[
            pl.BlockSpec((1000, D_FEAT), lambda i: (i, 0)),
            pl.BlockSpec((1000, D_FEAT), lambda i: (i, 0)),
        ],
        out_specs=pl.BlockSpec((1000, D_FEAT), lambda i: (i, 0)),
        out_shape=jax.ShapeDtypeStruct((NUM_SEGMENTS, D_FEAT), jnp.float32),
    )(a, b)


@jax.jit
def kernel(X, keys):
    keys = keys.astype(jnp.int32)
    acc = _sc_partial_sums(X, keys)
    return _tc_combine(acc[0], acc[1])
